# Initial kernel scaffold; baseline (speedup 1.0000x reference)
import jax
import jax.numpy as jnp
from jax.experimental import pallas as pl


def kernel(table, user_ids, item_ids):
    raise NotImplementedError("write your pallas kernel here")


# speedup vs baseline: 135.7911x; 135.7911x over previous
import functools

import jax
import jax.numpy as jnp
from jax import lax
from jax.experimental import pallas as pl
from jax.experimental.pallas import tpu as pltpu
from jax.experimental.pallas import tpu_sc as plsc

_INFO = plsc.get_sparse_core_info()
_NC = _INFO.num_cores
_NS = _INFO.num_subcores
_NW = _NC * _NS

_B = 16384 * 200
_B_PER_W = _B // _NW
_CHUNK = 12800
_STEPS = _B_PER_W // _CHUNK


def _gather_body(table_hbm, idx_hbm, out_hbm, idx_v, rows_v, sem):
    wid = lax.axis_index("s") * _NC + lax.axis_index("c")
    base = wid * _B_PER_W

    def step(i, carry):
        off = base + i * _CHUNK
        pltpu.sync_copy(idx_hbm.at[pl.ds(off, _CHUNK)], idx_v)
        pltpu.async_copy(table_hbm.at[idx_v], rows_v, sem).wait()
        pltpu.sync_copy(rows_v, out_hbm.at[pl.ds(off, _CHUNK)])
        return carry

    lax.fori_loop(0, _STEPS, step, 0)


@jax.jit
def _sc_gather(table, idx_flat):
    mesh = plsc.VectorSubcoreMesh(core_axis_name="c", subcore_axis_name="s")
    f = pl.kernel(
        _gather_body,
        mesh=mesh,
        out_type=jax.ShapeDtypeStruct((_B,), jnp.float32),
        scratch_types=[
            pltpu.VMEM((_CHUNK,), jnp.int32),
            pltpu.VMEM((_CHUNK,), jnp.float32),
            pltpu.SemaphoreType.DMA,
        ],
    )
    return f(table, idx_flat)


def kernel(table, user_ids, item_ids):
    idx_flat = item_ids.reshape(-1).astype(jnp.int32)
    out = _sc_gather(table, idx_flat)
    return out.reshape(item_ids.shape)


# speedup vs baseline: 213.2001x; 1.5701x over previous
import jax
import jax.numpy as jnp
from jax import lax
from jax.experimental import pallas as pl
from jax.experimental.pallas import tpu as pltpu
from jax.experimental.pallas import tpu_sc as plsc

_INFO = plsc.get_sparse_core_info()
_NC = _INFO.num_cores
_NS = _INFO.num_subcores
_NW = _NC * _NS

_VPAD = 1 << 20
_B = 16384 * 200
_B_PER_W = _B // _NW
_CHUNK = 12800
_STEPS = _B_PER_W // _CHUNK
_TAB_SLICE = _VPAD // _NS


def _gather_body(table_hbm, idx_hbm, out_hbm, tab_s, idx_v, rows_v, sem):
    cid = lax.axis_index("c")
    sid = lax.axis_index("s")
    wid = sid * _NC + cid
    base = wid * _B_PER_W

    tb = sid * _TAB_SLICE
    pltpu.sync_copy(table_hbm.at[pl.ds(tb, _TAB_SLICE)],
                    tab_s.at[pl.ds(tb, _TAB_SLICE)])
    plsc.subcore_barrier()

    def step(i, carry):
        off = base + i * _CHUNK
        pltpu.sync_copy(idx_hbm.at[pl.ds(off, _CHUNK)], idx_v)
        pltpu.async_copy(tab_s.at[idx_v], rows_v, sem).wait()
        pltpu.sync_copy(rows_v, out_hbm.at[pl.ds(off, _CHUNK)])
        return carry

    lax.fori_loop(0, _STEPS, step, 0)


@jax.jit
def _sc_gather(table_padded, idx_flat):
    mesh = plsc.VectorSubcoreMesh(core_axis_name="c", subcore_axis_name="s")
    f = pl.kernel(
        _gather_body,
        mesh=mesh,
        out_type=jax.ShapeDtypeStruct((_B,), jnp.float32),
        scratch_types=[
            pltpu.VMEM_SHARED((_VPAD,), jnp.float32),
            pltpu.VMEM((_CHUNK,), jnp.int32),
            pltpu.VMEM((_CHUNK,), jnp.float32),
            pltpu.SemaphoreType.DMA,
        ],
    )
    return f(table_padded, idx_flat)


def kernel(table, user_ids, item_ids):
    table_padded = jnp.pad(table, (0, _VPAD - table.shape[0]))
    idx_flat = item_ids.reshape(-1).astype(jnp.int32)
    out = _sc_gather(table_padded, idx_flat)
    return out.reshape(item_ids.shape)


# speedup vs baseline: 235.1641x; 1.1030x over previous
import jax
import jax.numpy as jnp
from jax import lax
from jax.experimental import pallas as pl
from jax.experimental.pallas import tpu as pltpu
from jax.experimental.pallas import tpu_sc as plsc

_INFO = plsc.get_sparse_core_info()
_NC = _INFO.num_cores
_NS = _INFO.num_subcores
_NW = _NC * _NS

_VPAD = 1 << 20
_B = 16384 * 200
_B_PER_W = _B // _NW
_CHUNK = 12800
_STEPS = _B_PER_W // _CHUNK
_TAB_SLICE = _VPAD // _NS


def _gather_body(table_hbm, idx_hbm, out_hbm, tab_s,
                 idx0, idx1, rows0, rows1,
                 sl0, sl1, sg0, sg1, ss0, ss1):
    cid = lax.axis_index("c")
    sid = lax.axis_index("s")
    wid = sid * _NC + cid
    base = wid * _B_PER_W

    idx_v = (idx0, idx1)
    rows_v = (rows0, rows1)
    sem_l = (sl0, sl1)
    sem_g = (sg0, sg1)
    sem_s = (ss0, ss1)

    def load(i):
        off = base + i * _CHUNK
        return pltpu.async_copy(idx_hbm.at[pl.ds(off, _CHUNK)],
                                idx_v[i % 2], sem_l[i % 2])

    def gather(i):
        return pltpu.async_copy(tab_s.at[idx_v[i % 2]],
                                rows_v[i % 2], sem_g[i % 2])

    def store(i):
        off = base + i * _CHUNK
        return pltpu.async_copy(rows_v[i % 2],
                                out_hbm.at[pl.ds(off, _CHUNK)], sem_s[i % 2])

    dma_l = {0: load(0)}
    dma_g, dma_s = {}, {}

    tb = sid * _TAB_SLICE
    pltpu.sync_copy(table_hbm.at[pl.ds(tb, _TAB_SLICE)],
                    tab_s.at[pl.ds(tb, _TAB_SLICE)])
    plsc.subcore_barrier()

    for i in range(_STEPS):
        dma_l[i].wait()
        if i >= 2:
            dma_s[i - 2].wait()
        dma_g[i] = gather(i)
        if i >= 1:
            dma_g[i - 1].wait()
            dma_s[i - 1] = store(i - 1)
        if i + 1 < _STEPS:
            dma_l[i + 1] = load(i + 1)
    dma_g[_STEPS - 1].wait()
    dma_s[_STEPS - 1] = store(_STEPS - 1)
    if _STEPS >= 2:
        dma_s[_STEPS - 2].wait()
    dma_s[_STEPS - 1].wait()


@jax.jit
def _sc_gather(table_padded, idx_flat):
    mesh = plsc.VectorSubcoreMesh(core_axis_name="c", subcore_axis_name="s")
    f = pl.kernel(
        _gather_body,
        mesh=mesh,
        out_type=jax.ShapeDtypeStruct((_B,), jnp.float32),
        scratch_types=[
            pltpu.VMEM_SHARED((_VPAD,), jnp.float32),
            pltpu.VMEM((_CHUNK,), jnp.int32),
            pltpu.VMEM((_CHUNK,), jnp.int32),
            pltpu.VMEM((_CHUNK,), jnp.float32),
            pltpu.VMEM((_CHUNK,), jnp.float32),
            pltpu.SemaphoreType.DMA,
            pltpu.SemaphoreType.DMA,
            pltpu.SemaphoreType.DMA,
            pltpu.SemaphoreType.DMA,
            pltpu.SemaphoreType.DMA,
            pltpu.SemaphoreType.DMA,
        ],
    )
    return f(table_padded, idx_flat)


def kernel(table, user_ids, item_ids):
    table_padded = jnp.pad(table, (0, _VPAD - table.shape[0]))
    idx_flat = item_ids.reshape(-1).astype(jnp.int32)
    out = _sc_gather(table_padded, idx_flat)
    return out.reshape(item_ids.shape)
